# speedup vs baseline: 15.1159x; 15.1159x over previous
import functools

import jax
import jax.numpy as jnp
from jax import lax
from jax.experimental import pallas as pl
from jax.experimental.pallas import tpu as pltpu
from jax.experimental.pallas import tpu_sc as plsc

N = 10000
D = 128
E = 320000

NC = 2
NS = 16
NW = NC * NS
EPT = E // NW
K = 80
CH = EPT // K
N_PAD = 10240
RPT = N_PAD // NS

_mesh = plsc.VectorSubcoreMesh(core_axis_name="c", subcore_axis_name="s")


@functools.partial(
    pl.kernel,
    out_type=jax.ShapeDtypeStruct((NC, N_PAD), jnp.float32),
    mesh=_mesh,
    scratch_types=[
        pltpu.VMEM((CH, K), jnp.int32),
        pltpu.VMEM((K,), jnp.float32),
        pltpu.VMEM((RPT,), jnp.float32),
        pltpu.VMEM_SHARED((N_PAD,), jnp.float32),
    ],
)
def _deg_parts(dst_hbm, out_hbm, dst_v, ones_v, zero_v, acc_sh):
    c = lax.axis_index("c")
    s = lax.axis_index("s")
    wid = s * NC + c

    one16 = jnp.ones((16,), jnp.float32)
    zero16 = jnp.zeros((16,), jnp.float32)
    for i in range(K // 16):
        ones_v[pl.ds(i * 16, 16)] = one16

    def _z(i, _):
        zero_v[pl.ds(i * 16, 16)] = zero16
        return 0

    lax.fori_loop(0, RPT // 16, _z, 0)

    pltpu.sync_copy(zero_v, acc_sh.at[pl.ds(s * RPT, RPT)])
    pltpu.sync_copy(dst_hbm.at[wid], dst_v)
    plsc.subcore_barrier()

    def _chunk(j, _):
        pltpu.sync_copy(ones_v, acc_sh.at[dst_v.at[j]], add=True)
        return 0

    lax.fori_loop(0, CH, _chunk, 0)
    plsc.subcore_barrier()
    pltpu.sync_copy(acc_sh.at[pl.ds(s * RPT, RPT)],
                    out_hbm.at[c, pl.ds(s * RPT, RPT)])


@functools.partial(
    pl.kernel,
    out_type=jax.ShapeDtypeStruct((NC, N_PAD, D), jnp.float32),
    mesh=_mesh,
    scratch_types=[
        pltpu.VMEM((CH, K), jnp.int32),
        pltpu.VMEM((CH, K), jnp.int32),
        pltpu.VMEM((K, D), jnp.float32),
        pltpu.VMEM_SHARED((N_PAD, D), jnp.float32),
        pltpu.SemaphoreType.DMA,
    ],
)
def _scatter_parts(hp_hbm, src_hbm, dst_hbm, zeros_hbm, out_hbm,
                   src_v, dst_v, rows_v, acc_sh, sem):
    c = lax.axis_index("c")
    s = lax.axis_index("s")
    wid = s * NC + c
    rbase = s * RPT

    @pl.when(c == 0)
    def _():
        pltpu.sync_copy(hp_hbm.at[pl.ds(rbase, RPT)],
                        acc_sh.at[pl.ds(rbase, RPT)])

    @pl.when(c != 0)
    def _():
        pltpu.sync_copy(zeros_hbm.at[pl.ds(rbase, RPT)],
                        acc_sh.at[pl.ds(rbase, RPT)])

    pltpu.sync_copy(src_hbm.at[wid], src_v)
    pltpu.sync_copy(dst_hbm.at[wid], dst_v)
    plsc.subcore_barrier()

    def _chunk(j, _):
        pltpu.async_copy(hp_hbm.at[src_v.at[j]], rows_v, sem).wait()
        pltpu.sync_copy(rows_v, acc_sh.at[dst_v.at[j]], add=True)
        return 0

    lax.fori_loop(0, CH, _chunk, 0)
    plsc.subcore_barrier()
    pltpu.sync_copy(acc_sh.at[pl.ds(rbase, RPT)],
                    out_hbm.at[c, pl.ds(rbase, RPT)])


BR = 256
_GRID = (N_PAD // BR,)


def _dinv_of(degp_blk):
    return lax.rsqrt(1.0 + jnp.sum(degp_blk, axis=1, keepdims=True))


def _tc1_body(x_ref, w_ref, degp_ref, hp_ref):
    dinv = _dinv_of(degp_ref[...])
    h = jnp.dot(x_ref[...], w_ref[...], preferred_element_type=jnp.float32)
    hp_ref[...] = dinv * h


def _tcmid_body(p0_ref, p1_ref, degp_ref, b_ref, w_ref, hp_ref):
    dinv = _dinv_of(degp_ref[...])
    z = jnp.maximum(dinv * (p0_ref[...] + p1_ref[...]) + b_ref[...], 0.0)
    h = jnp.dot(z, w_ref[...], preferred_element_type=jnp.float32)
    hp_ref[...] = dinv * h


def _tcfin_body(p0_ref, p1_ref, degp_ref, b_ref, out_ref):
    dinv = _dinv_of(degp_ref[...])
    out_ref[...] = dinv * (p0_ref[...] + p1_ref[...]) + b_ref[...]


_row_spec = pl.BlockSpec((BR, D), lambda i: (i, 0))
_degp_spec = pl.BlockSpec((BR, NC), lambda i: (i, 0))
_w_spec = pl.BlockSpec((D, D), lambda i: (0, 0))
_b_spec = pl.BlockSpec((1, D), lambda i: (0, 0))
_out_t = jax.ShapeDtypeStruct((N_PAD, D), jnp.float32)

_tc1 = pl.pallas_call(
    _tc1_body, grid=_GRID,
    in_specs=[_row_spec, _w_spec, _degp_spec],
    out_specs=_row_spec, out_shape=_out_t)

_tcmid = pl.pallas_call(
    _tcmid_body, grid=_GRID,
    in_specs=[_row_spec, _row_spec, _degp_spec, _b_spec, _w_spec],
    out_specs=_row_spec, out_shape=_out_t)

_tcfin = pl.pallas_call(
    _tcfin_body, grid=_GRID,
    in_specs=[_row_spec, _row_spec, _degp_spec, _b_spec],
    out_specs=_row_spec, out_shape=_out_t)


def kernel(x, edge_index, W1, b1, W2, b2, W3, b3):
    src = edge_index[0].astype(jnp.int32).reshape(NW, CH, K)
    dst = edge_index[1].astype(jnp.int32).reshape(NW, CH, K)
    x_pad = jnp.zeros((N_PAD, D), jnp.float32).at[:N].set(x)
    zeros2 = jnp.zeros((N_PAD, D), jnp.float32)

    degp = _deg_parts(dst).T

    hp = _tc1(x_pad, W1, degp)
    parts = _scatter_parts(hp, src, dst, zeros2)
    hp = _tcmid(parts[0], parts[1], degp, b1.reshape(1, D), W2)
    parts = _scatter_parts(hp, src, dst, zeros2)
    hp = _tcmid(parts[0], parts[1], degp, b2.reshape(1, D), W3)
    parts = _scatter_parts(hp, src, dst, zeros2)
    out = _tcfin(parts[0], parts[1], degp, b3.reshape(1, D))
    return out[:N]


# speedup vs baseline: 22.4055x; 1.4822x over previous
import functools

import jax
import jax.numpy as jnp
from jax import lax
from jax.experimental import pallas as pl
from jax.experimental.pallas import tpu as pltpu
from jax.experimental.pallas import tpu_sc as plsc

N = 10000
D = 128
E = 320000

NC = 2
NS = 16
NW = NC * NS
EPT = E // NW
K = 80
CH = EPT // K
NBUF = 2
PD = 1
N_PAD = 10240
RPT = N_PAD // NS

_mesh = plsc.VectorSubcoreMesh(core_axis_name="c", subcore_axis_name="s")


@functools.partial(
    pl.kernel,
    out_type=jax.ShapeDtypeStruct((NC, N_PAD), jnp.float32),
    mesh=_mesh,
    scratch_types=[
        pltpu.VMEM((CH, K), jnp.int32),
        pltpu.VMEM((128,), jnp.float32),
        pltpu.VMEM((RPT,), jnp.float32),
        pltpu.VMEM_SHARED((N_PAD,), jnp.float32),
    ],
)
def _deg_parts(dst_hbm, out_hbm, dst_v, ones_v, zero_v, acc_sh):
    c = lax.axis_index("c")
    s = lax.axis_index("s")
    wid = s * NC + c

    one16 = jnp.ones((16,), jnp.float32)
    zero16 = jnp.zeros((16,), jnp.float32)
    for i in range(8):
        ones_v[pl.ds(i * 16, 16)] = one16

    def _z(i, _):
        zero_v[pl.ds(i * 16, 16)] = zero16
        return 0

    lax.fori_loop(0, RPT // 16, _z, 0)

    pltpu.sync_copy(zero_v, acc_sh.at[pl.ds(s * RPT, RPT)])
    pltpu.sync_copy(dst_hbm.at[wid], dst_v)
    plsc.subcore_barrier()

    def _chunk(j, _):
        pltpu.sync_copy(ones_v.at[pl.ds(0, K)], acc_sh.at[dst_v.at[j]],
                        add=True)
        return 0

    lax.fori_loop(0, CH, _chunk, 0)
    plsc.subcore_barrier()
    pltpu.sync_copy(acc_sh.at[pl.ds(s * RPT, RPT)],
                    out_hbm.at[c, pl.ds(s * RPT, RPT)])


@functools.partial(
    pl.kernel,
    out_type=jax.ShapeDtypeStruct((NC, N_PAD, D), jnp.float32),
    mesh=_mesh,
    scratch_types=[
        pltpu.VMEM((EPT,), jnp.int32),
        pltpu.VMEM((CH, K), jnp.int32),
        pltpu.VMEM((NBUF, K, D), jnp.float32),
        pltpu.VMEM_SHARED((N_PAD, D), jnp.float32),
        pltpu.SemaphoreType.DMA((NBUF,)),
        pltpu.SemaphoreType.DMA((NBUF,)),
    ],
)
def _scatter_parts(hp_hbm, src_hbm, dst_hbm, zeros_hbm, out_hbm,
                   src_v, dst_v, rows_v, acc_sh, gsem, ssem):
    c = lax.axis_index("c")
    s = lax.axis_index("s")
    wid = s * NC + c
    rbase = s * RPT

    @pl.when(c == 0)
    def _():
        pltpu.sync_copy(hp_hbm.at[pl.ds(rbase, RPT)],
                        acc_sh.at[pl.ds(rbase, RPT)])

    @pl.when(c != 0)
    def _():
        pltpu.sync_copy(zeros_hbm.at[pl.ds(rbase, RPT)],
                        acc_sh.at[pl.ds(rbase, RPT)])

    pltpu.sync_copy(src_hbm.at[wid], src_v)
    pltpu.sync_copy(dst_hbm.at[wid], dst_v)
    plsc.subcore_barrier()

    def _src_idx(g):
        return src_v.at[pl.ds(pl.multiple_of(g * K, K), K)]

    def _gather_start(g, b):
        pltpu.async_copy(hp_hbm.at[_src_idx(g)], rows_v.at[b], gsem.at[b])

    def _gather_wait(g, b):
        pltpu.make_async_copy(hp_hbm.at[_src_idx(g)], rows_v.at[b],
                              gsem.at[b]).wait()

    def _scatter_start(g, b):
        pltpu.async_copy(rows_v.at[b], acc_sh.at[dst_v.at[g]], ssem.at[b],
                         add=True)

    def _scatter_wait(g, b):
        pltpu.make_async_copy(rows_v.at[b], acc_sh.at[dst_v.at[g]],
                              ssem.at[b]).wait()

    HEAD = NBUF - PD
    HEADX = (CH - PD - HEAD) % NBUF
    for g in range(PD):
        _gather_start(g, g % NBUF)
    for g in range(HEAD):
        _gather_start(g + PD, (g + PD) % NBUF)
        _gather_wait(g, g % NBUF)
        _scatter_start(g, g % NBUF)
    for g in range(HEAD, HEAD + HEADX):
        _scatter_wait(g - HEAD, (g + PD) % NBUF)
        _gather_start(g + PD, (g + PD) % NBUF)
        _gather_wait(g, g % NBUF)
        _scatter_start(g, g % NBUF)

    def _body(t, _):
        o = HEAD + HEADX + NBUF * t
        for i in range(NBUF):
            g = o + i
            gm = (HEAD + HEADX + i) % NBUF
            bp = (gm + PD) % NBUF
            _scatter_wait(g - HEAD, bp)
            _gather_start(g + PD, bp)
            _gather_wait(g, gm)
            _scatter_start(g, gm)
        return 0

    lax.fori_loop(0, (CH - PD - HEAD - HEADX) // NBUF, _body, 0)
    for g in range(CH - PD, CH):
        _gather_wait(g, g % NBUF)
        _scatter_start(g, g % NBUF)
    for g in range(CH - NBUF, CH):
        _scatter_wait(g, g % NBUF)

    plsc.subcore_barrier()
    pltpu.sync_copy(acc_sh.at[pl.ds(rbase, RPT)],
                    out_hbm.at[c, pl.ds(rbase, RPT)])


BR = 256
_GRID = (N_PAD // BR,)


def _dinv_of(degp_blk):
    return lax.rsqrt(1.0 + jnp.sum(degp_blk, axis=1, keepdims=True))


def _tc1_body(x_ref, w_ref, degp_ref, hp_ref):
    dinv = _dinv_of(degp_ref[...])
    h = jnp.dot(x_ref[...], w_ref[...], preferred_element_type=jnp.float32)
    hp_ref[...] = dinv * h


def _tcmid_body(p0_ref, p1_ref, degp_ref, b_ref, w_ref, hp_ref):
    dinv = _dinv_of(degp_ref[...])
    z = jnp.maximum(dinv * (p0_ref[...] + p1_ref[...]) + b_ref[...], 0.0)
    h = jnp.dot(z, w_ref[...], preferred_element_type=jnp.float32)
    hp_ref[...] = dinv * h


def _tcfin_body(p0_ref, p1_ref, degp_ref, b_ref, out_ref):
    dinv = _dinv_of(degp_ref[...])
    out_ref[...] = dinv * (p0_ref[...] + p1_ref[...]) + b_ref[...]


_row_spec = pl.BlockSpec((BR, D), lambda i: (i, 0))
_degp_spec = pl.BlockSpec((BR, NC), lambda i: (i, 0))
_w_spec = pl.BlockSpec((D, D), lambda i: (0, 0))
_b_spec = pl.BlockSpec((1, D), lambda i: (0, 0))
_out_t = jax.ShapeDtypeStruct((N_PAD, D), jnp.float32)

_tc1 = pl.pallas_call(
    _tc1_body, grid=_GRID,
    in_specs=[_row_spec, _w_spec, _degp_spec],
    out_specs=_row_spec, out_shape=_out_t)

_tcmid = pl.pallas_call(
    _tcmid_body, grid=_GRID,
    in_specs=[_row_spec, _row_spec, _degp_spec, _b_spec, _w_spec],
    out_specs=_row_spec, out_shape=_out_t)

_tcfin = pl.pallas_call(
    _tcfin_body, grid=_GRID,
    in_specs=[_row_spec, _row_spec, _degp_spec, _b_spec],
    out_specs=_row_spec, out_shape=_out_t)


def kernel(x, edge_index, W1, b1, W2, b2, W3, b3):
    src = edge_index[0].astype(jnp.int32).reshape(NW, EPT)
    dst = edge_index[1].astype(jnp.int32).reshape(NW, CH, K)
    x_pad = jnp.zeros((N_PAD, D), jnp.float32).at[:N].set(x)
    zeros2 = jnp.zeros((N_PAD, D), jnp.float32)

    degp = _deg_parts(dst).T

    hp = _tc1(x_pad, W1, degp)
    parts = _scatter_parts(hp, src, dst, zeros2)
    hp = _tcmid(parts[0], parts[1], degp, b1.reshape(1, D), W2)
    parts = _scatter_parts(hp, src, dst, zeros2)
    hp = _tcmid(parts[0], parts[1], degp, b2.reshape(1, D), W3)
    parts = _scatter_parts(hp, src, dst, zeros2)
    out = _tcfin(parts[0], parts[1], degp, b3.reshape(1, D))
    return out[:N]


# speedup vs baseline: 23.1029x; 1.0311x over previous
import functools

import jax
import jax.numpy as jnp
from jax import lax
from jax.experimental import pallas as pl
from jax.experimental.pallas import tpu as pltpu
from jax.experimental.pallas import tpu_sc as plsc

N = 10000
D = 128
E = 320000

NC = 2
NS = 16
NW = NC * NS
EPT = E // NW
K = 128
SP = 10240
CH = SP // K
NBUF = 2
SNB = 4
N_PAD = 10240
RPT = N_PAD // NS

_mesh = plsc.VectorSubcoreMesh(core_axis_name="c", subcore_axis_name="s")


@functools.partial(
    pl.kernel,
    out_type=jax.ShapeDtypeStruct((NC, N_PAD), jnp.float32),
    mesh=_mesh,
    scratch_types=[
        pltpu.VMEM((CH, K), jnp.int32),
        pltpu.VMEM((K,), jnp.float32),
        pltpu.VMEM((RPT,), jnp.float32),
        pltpu.VMEM_SHARED((N_PAD,), jnp.float32),
    ],
)
def _deg_parts(dst_hbm, out_hbm, dst_v, ones_v, zero_v, acc_sh):
    c = lax.axis_index("c")
    s = lax.axis_index("s")
    wid = s * NC + c

    one16 = jnp.ones((16,), jnp.float32)
    zero16 = jnp.zeros((16,), jnp.float32)
    for i in range(K // 16):
        ones_v[pl.ds(i * 16, 16)] = one16

    def _z(i, _):
        zero_v[pl.ds(i * 16, 16)] = zero16
        return 0

    lax.fori_loop(0, RPT // 16, _z, 0)

    pltpu.sync_copy(zero_v, acc_sh.at[pl.ds(s * RPT, RPT)])
    pltpu.sync_copy(dst_hbm.at[wid], dst_v)
    plsc.subcore_barrier()

    def _chunk(j, _):
        pltpu.sync_copy(ones_v, acc_sh.at[dst_v.at[j]], add=True)
        return 0

    lax.fori_loop(0, CH, _chunk, 0)
    plsc.subcore_barrier()
    pltpu.sync_copy(acc_sh.at[pl.ds(s * RPT, RPT)],
                    out_hbm.at[c, pl.ds(s * RPT, RPT)])


@functools.partial(
    pl.kernel,
    out_type=jax.ShapeDtypeStruct((NC, N_PAD, D), jnp.float32),
    mesh=_mesh,
    scratch_types=[
        pltpu.VMEM((SNB, K), jnp.int32),
        pltpu.VMEM((CH, K), jnp.int32),
        pltpu.VMEM((NBUF, K, D), jnp.float32),
        pltpu.VMEM_SHARED((N_PAD, D), jnp.float32),
        pltpu.SemaphoreType.DMA((NBUF,)),
        pltpu.SemaphoreType.DMA((NBUF,)),
        pltpu.SemaphoreType.DMA((SNB,)),
    ],
)
def _scatter_parts(hp_hbm, src_hbm, dst_hbm, zeros_hbm, out_hbm,
                   src_v, dst_v, rows_v, acc_sh, gsem, ssem, xsem):
    c = lax.axis_index("c")
    s = lax.axis_index("s")
    wid = s * NC + c
    rbase = s * RPT

    @pl.when(c == 0)
    def _():
        pltpu.sync_copy(hp_hbm.at[pl.ds(rbase, RPT)],
                        acc_sh.at[pl.ds(rbase, RPT)])

    @pl.when(c != 0)
    def _():
        pltpu.sync_copy(zeros_hbm.at[pl.ds(rbase, RPT)],
                        acc_sh.at[pl.ds(rbase, RPT)])

    pltpu.sync_copy(dst_hbm.at[wid], dst_v)

    def _src_start(q):
        pltpu.async_copy(src_hbm.at[wid, q], src_v.at[q % SNB],
                         xsem.at[q % SNB])

    def _src_wait(q):
        pltpu.make_async_copy(src_hbm.at[wid, q], src_v.at[q % SNB],
                              xsem.at[q % SNB]).wait()

    def _gather_start(g):
        pltpu.async_copy(hp_hbm.at[src_v.at[g % SNB]], rows_v.at[g % NBUF],
                         gsem.at[g % NBUF])

    def _gather_wait(g):
        pltpu.make_async_copy(hp_hbm.at[src_v.at[g % SNB]],
                              rows_v.at[g % NBUF], gsem.at[g % NBUF]).wait()

    def _scatter_start(g):
        pltpu.async_copy(rows_v.at[g % NBUF], acc_sh.at[dst_v.at[g]],
                         ssem.at[g % NBUF], add=True)

    def _scatter_wait(g):
        pltpu.make_async_copy(rows_v.at[g % NBUF], acc_sh.at[dst_v.at[g]],
                              ssem.at[g % NBUF]).wait()

    for q in range(3):
        _src_start(q)
    _src_wait(0)
    _gather_start(0)
    plsc.subcore_barrier()

    _src_start(3)
    _src_wait(1)
    _gather_start(1)
    _gather_wait(0)
    _scatter_start(0)

    def _body(t, _):
        o = 1 + 4 * t
        for i in range(4):
            g = o + i
            _src_start(g + 3)
            _scatter_wait(g - 1)
            _src_wait(g + 1)
            _gather_start(g + 1)
            _gather_wait(g)
            _scatter_start(g)
        return 0

    lax.fori_loop(0, (CH - 4) // 4, _body, 0)
    for g in range(CH - 3, CH - 1):
        _scatter_wait(g - 1)
        _src_wait(g + 1)
        _gather_start(g + 1)
        _gather_wait(g)
        _scatter_start(g)
    g = CH - 1
    _scatter_wait(g - 1)
    _gather_wait(g)
    _scatter_start(g)
    _scatter_wait(g)

    plsc.subcore_barrier()
    pltpu.sync_copy(acc_sh.at[pl.ds(rbase, RPT)],
                    out_hbm.at[c, pl.ds(rbase, RPT)])

    plsc.subcore_barrier()
    pltpu.sync_copy(acc_sh.at[pl.ds(rbase, RPT)],
                    out_hbm.at[c, pl.ds(rbase, RPT)])


BR = 256
_GRID = (N_PAD // BR,)


def _dinv_of(degp_blk):
    return lax.rsqrt(1.0 + jnp.sum(degp_blk, axis=1, keepdims=True))


def _tc1_body(x_ref, w_ref, degp_ref, hp_ref):
    dinv = _dinv_of(degp_ref[...])
    h = jnp.dot(x_ref[...], w_ref[...], preferred_element_type=jnp.float32)
    hp_ref[...] = dinv * h


def _tcmid_body(p0_ref, p1_ref, degp_ref, b_ref, w_ref, hp_ref):
    dinv = _dinv_of(degp_ref[...])
    z = jnp.maximum(dinv * (p0_ref[...] + p1_ref[...]) + b_ref[...], 0.0)
    h = jnp.dot(z, w_ref[...], preferred_element_type=jnp.float32)
    hp_ref[...] = dinv * h


def _tcfin_body(p0_ref, p1_ref, degp_ref, b_ref, out_ref):
    dinv = _dinv_of(degp_ref[...])
    out_ref[...] = dinv * (p0_ref[...] + p1_ref[...]) + b_ref[...]


_row_spec = pl.BlockSpec((BR, D), lambda i: (i, 0))
_degp_spec = pl.BlockSpec((BR, NC), lambda i: (i, 0))
_w_spec = pl.BlockSpec((D, D), lambda i: (0, 0))
_b_spec = pl.BlockSpec((1, D), lambda i: (0, 0))
_out_t = jax.ShapeDtypeStruct((N_PAD, D), jnp.float32)

_tc1 = pl.pallas_call(
    _tc1_body, grid=_GRID,
    in_specs=[_row_spec, _w_spec, _degp_spec],
    out_specs=_row_spec, out_shape=_out_t)

_tcmid = pl.pallas_call(
    _tcmid_body, grid=_GRID,
    in_specs=[_row_spec, _row_spec, _degp_spec, _b_spec, _w_spec],
    out_specs=_row_spec, out_shape=_out_t)

_tcfin = pl.pallas_call(
    _tcfin_body, grid=_GRID,
    in_specs=[_row_spec, _row_spec, _degp_spec, _b_spec],
    out_specs=_row_spec, out_shape=_out_t)


def kernel(x, edge_index, W1, b1, W2, b2, W3, b3):
    pad = jnp.broadcast_to(jnp.arange(N, N_PAD, dtype=jnp.int32),
                           (NW, SP - EPT))
    src = jnp.concatenate(
        [edge_index[0].astype(jnp.int32).reshape(NW, EPT), pad],
        axis=1).reshape(NW, CH, K)
    dst = jnp.concatenate(
        [edge_index[1].astype(jnp.int32).reshape(NW, EPT), pad],
        axis=1).reshape(NW, CH, K)
    x_pad = jnp.zeros((N_PAD, D), jnp.float32).at[:N].set(x)
    zeros2 = jnp.zeros((N_PAD, D), jnp.float32)

    degp = _deg_parts(dst).T

    hp = _tc1(x_pad, W1, degp)
    parts = _scatter_parts(hp, src, dst, zeros2)
    hp = _tcmid(parts[0], parts[1], degp, b1.reshape(1, D), W2)
    parts = _scatter_parts(hp, src, dst, zeros2)
    hp = _tcmid(parts[0], parts[1], degp, b2.reshape(1, D), W3)
    parts = _scatter_parts(hp, src, dst, zeros2)
    out = _tcfin(parts[0], parts[1], degp, b3.reshape(1, D))
    return out[:N]


# speedup vs baseline: 23.6661x; 1.0244x over previous
import functools

import jax
import jax.numpy as jnp
from jax import lax
from jax.experimental import pallas as pl
from jax.experimental.pallas import tpu as pltpu
from jax.experimental.pallas import tpu_sc as plsc

N = 10000
D = 128
E = 320000

NC = 2
NS = 16
NW = NC * NS
EPT = E // NW
K = 80
SP = 10240
CH = SP // K
NBUF = 4
SNB = 6
N_PAD = 10240
RPT = N_PAD // NS

_mesh = plsc.VectorSubcoreMesh(core_axis_name="c", subcore_axis_name="s")


@functools.partial(
    pl.kernel,
    out_type=jax.ShapeDtypeStruct((NC, N_PAD), jnp.float32),
    mesh=_mesh,
    scratch_types=[
        pltpu.VMEM((CH, K), jnp.int32),
        pltpu.VMEM((K,), jnp.float32),
        pltpu.VMEM((RPT,), jnp.float32),
        pltpu.VMEM_SHARED((N_PAD,), jnp.float32),
    ],
)
def _deg_parts(dst_hbm, out_hbm, dst_v, ones_v, zero_v, acc_sh):
    c = lax.axis_index("c")
    s = lax.axis_index("s")
    wid = s * NC + c

    one16 = jnp.ones((16,), jnp.float32)
    zero16 = jnp.zeros((16,), jnp.float32)
    for i in range(K // 16):
        ones_v[pl.ds(i * 16, 16)] = one16

    def _z(i, _):
        zero_v[pl.ds(i * 16, 16)] = zero16
        return 0

    lax.fori_loop(0, RPT // 16, _z, 0)

    pltpu.sync_copy(zero_v, acc_sh.at[pl.ds(s * RPT, RPT)])
    pltpu.sync_copy(dst_hbm.at[wid], dst_v)
    plsc.subcore_barrier()

    def _chunk(j, _):
        pltpu.sync_copy(ones_v, acc_sh.at[dst_v.at[j]], add=True)
        return 0

    lax.fori_loop(0, CH, _chunk, 0)
    plsc.subcore_barrier()
    pltpu.sync_copy(acc_sh.at[pl.ds(s * RPT, RPT)],
                    out_hbm.at[c, pl.ds(s * RPT, RPT)])


@functools.partial(
    pl.kernel,
    out_type=jax.ShapeDtypeStruct((NC, N_PAD, D), jnp.float32),
    mesh=_mesh,
    scratch_types=[
        pltpu.VMEM((SNB, K), jnp.int32),
        pltpu.VMEM((SNB, K), jnp.int32),
        pltpu.VMEM((NBUF, K, D), jnp.float32),
        pltpu.VMEM_SHARED((N_PAD, D), jnp.float32),
        pltpu.SemaphoreType.DMA((NBUF,)),
        pltpu.SemaphoreType.DMA((NBUF,)),
        pltpu.SemaphoreType.DMA((SNB,)),
        pltpu.SemaphoreType.DMA((SNB,)),
    ],
)
def _scatter_parts(hp_hbm, src_hbm, dst_hbm, zeros_hbm, out_hbm,
                   src_v, dst_v, rows_v, acc_sh, gsem, ssem, xsem, ysem):
    c = lax.axis_index("c")
    s = lax.axis_index("s")
    wid = s * NC + c
    rbase = s * RPT

    @pl.when(c == 0)
    def _():
        pltpu.sync_copy(hp_hbm.at[pl.ds(rbase, RPT)],
                        acc_sh.at[pl.ds(rbase, RPT)])

    @pl.when(c != 0)
    def _():
        pltpu.sync_copy(zeros_hbm.at[pl.ds(rbase, RPT)],
                        acc_sh.at[pl.ds(rbase, RPT)])

    def _idx_start(q):
        pltpu.async_copy(src_hbm.at[wid, q], src_v.at[q % SNB],
                         xsem.at[q % SNB])
        pltpu.async_copy(dst_hbm.at[wid, q], dst_v.at[q % SNB],
                         ysem.at[q % SNB])

    def _idx_wait(q):
        pltpu.make_async_copy(src_hbm.at[wid, q], src_v.at[q % SNB],
                              xsem.at[q % SNB]).wait()
        pltpu.make_async_copy(dst_hbm.at[wid, q], dst_v.at[q % SNB],
                              ysem.at[q % SNB]).wait()

    def _gather_start(g):
        pltpu.async_copy(hp_hbm.at[src_v.at[g % SNB]], rows_v.at[g % NBUF],
                         gsem.at[g % NBUF])

    def _gather_wait(g):
        pltpu.make_async_copy(hp_hbm.at[src_v.at[g % SNB]],
                              rows_v.at[g % NBUF], gsem.at[g % NBUF]).wait()

    def _scatter_start(g):
        pltpu.async_copy(rows_v.at[g % NBUF], acc_sh.at[dst_v.at[g % SNB]],
                         ssem.at[g % NBUF], add=True)

    def _scatter_wait(g):
        pltpu.make_async_copy(rows_v.at[g % NBUF],
                              acc_sh.at[dst_v.at[g % SNB]],
                              ssem.at[g % NBUF]).wait()

    def _iter(g, scw, idx, gat):
        if scw:
            _scatter_wait(g - 2)
        if idx:
            _idx_start(g + 4)
        if gat:
            _idx_wait(g + 2)
            _gather_start(g + 2)
        _gather_wait(g)
        _scatter_start(g)

    for q in range(4):
        _idx_start(q)
    for g in range(2):
        _idx_wait(g)
        _gather_start(g)
    plsc.subcore_barrier()

    for g in range(2):
        _iter(g, False, True, True)
    for g in range(2, 4):
        _iter(g, True, True, True)

    def _body(t, _):
        o = 4 + 12 * t
        for i in range(12):
            _iter(o + i, True, True, True)
        return 0

    lax.fori_loop(0, (CH - 8) // 12, _body, 0)
    for g in range(CH - 4, CH - 2):
        _iter(g, True, False, True)
    for g in range(CH - 2, CH):
        _iter(g, True, False, False)
    for g in range(CH - 2, CH):
        _scatter_wait(g)

    plsc.subcore_barrier()
    pltpu.sync_copy(acc_sh.at[pl.ds(rbase, RPT)],
                    out_hbm.at[c, pl.ds(rbase, RPT)])

    plsc.subcore_barrier()
    pltpu.sync_copy(acc_sh.at[pl.ds(rbase, RPT)],
                    out_hbm.at[c, pl.ds(rbase, RPT)])


BR = 256
_GRID = (N_PAD // BR,)


def _dinv_of(degp_blk):
    return lax.rsqrt(1.0 + jnp.sum(degp_blk, axis=1, keepdims=True))


def _tc1_body(x_ref, w_ref, degp_ref, hp_ref):
    dinv = _dinv_of(degp_ref[...])
    h = jnp.dot(x_ref[...], w_ref[...], preferred_element_type=jnp.float32)
    hp_ref[...] = dinv * h


def _tcmid_body(p0_ref, p1_ref, degp_ref, b_ref, w_ref, hp_ref):
    dinv = _dinv_of(degp_ref[...])
    z = jnp.maximum(dinv * (p0_ref[...] + p1_ref[...]) + b_ref[...], 0.0)
    h = jnp.dot(z, w_ref[...], preferred_element_type=jnp.float32)
    hp_ref[...] = dinv * h


def _tcfin_body(p0_ref, p1_ref, degp_ref, b_ref, out_ref):
    dinv = _dinv_of(degp_ref[...])
    out_ref[...] = dinv * (p0_ref[...] + p1_ref[...]) + b_ref[...]


_row_spec = pl.BlockSpec((BR, D), lambda i: (i, 0))
_degp_spec = pl.BlockSpec((BR, NC), lambda i: (i, 0))
_w_spec = pl.BlockSpec((D, D), lambda i: (0, 0))
_b_spec = pl.BlockSpec((1, D), lambda i: (0, 0))
_out_t = jax.ShapeDtypeStruct((N_PAD, D), jnp.float32)

_tc1 = pl.pallas_call(
    _tc1_body, grid=_GRID,
    in_specs=[_row_spec, _w_spec, _degp_spec],
    out_specs=_row_spec, out_shape=_out_t)

_tcmid = pl.pallas_call(
    _tcmid_body, grid=_GRID,
    in_specs=[_row_spec, _row_spec, _degp_spec, _b_spec, _w_spec],
    out_specs=_row_spec, out_shape=_out_t)

_tcfin = pl.pallas_call(
    _tcfin_body, grid=_GRID,
    in_specs=[_row_spec, _row_spec, _degp_spec, _b_spec],
    out_specs=_row_spec, out_shape=_out_t)


def kernel(x, edge_index, W1, b1, W2, b2, W3, b3):
    pad = jnp.broadcast_to(jnp.arange(N, N_PAD, dtype=jnp.int32),
                           (NW, SP - EPT))
    src = jnp.concatenate(
        [edge_index[0].astype(jnp.int32).reshape(NW, EPT), pad],
        axis=1).reshape(NW, CH, K)
    dst = jnp.concatenate(
        [edge_index[1].astype(jnp.int32).reshape(NW, EPT), pad],
        axis=1).reshape(NW, CH, K)
    x_pad = jnp.zeros((N_PAD, D), jnp.float32).at[:N].set(x)
    zeros2 = jnp.zeros((N_PAD, D), jnp.float32)

    degp = _deg_parts(dst).T

    hp = _tc1(x_pad, W1, degp)
    parts = _scatter_parts(hp, src, dst, zeros2)
    hp = _tcmid(parts[0], parts[1], degp, b1.reshape(1, D), W2)
    parts = _scatter_parts(hp, src, dst, zeros2)
    hp = _tcmid(parts[0], parts[1], degp, b2.reshape(1, D), W3)
    parts = _scatter_parts(hp, src, dst, zeros2)
    out = _tcfin(parts[0], parts[1], degp, b3.reshape(1, D))
    return out[:N]
